# Initial kernel scaffold; baseline (speedup 1.0000x reference)
import jax
import jax.numpy as jnp
from jax.experimental import pallas as pl


def kernel(user_emb, entity_emb, edge_index, edge_type, inter_edge, inter_edge_w, relation_emb, W_Q):
    raise NotImplementedError("write your pallas kernel here")


# speedup vs baseline: 1.0124x; 1.0124x over previous
import math

import jax
import jax.numpy as jnp
from jax.experimental import pallas as pl

N_ENT = 10000
N_USR = 10000
CH = 128
N_HEADS = 2
D_K = CH // N_HEADS


def _scatter_softmax(src, index, num_segments):
    smax = jax.ops.segment_max(src, index, num_segments=num_segments)
    smax = jnp.where(jnp.isfinite(smax), smax, 0.0)
    shifted = src - smax[index]
    ex = jnp.exp(shifted)
    ssum = jax.ops.segment_sum(ex, index, num_segments=num_segments)
    return ex / (ssum[index] + 1e-16)


def _norm_res_kernel(agg_ref, res_ref, out_norm_ref, out_res_ref):
    x = agg_ref[...]
    n = jnp.sqrt(jnp.sum(x * x, axis=1, keepdims=True))
    xn = x / jnp.maximum(n, 1e-12)
    out_norm_ref[...] = xn
    out_res_ref[...] = res_ref[...] + xn


def _norm_res(agg, res):
    return pl.pallas_call(
        _norm_res_kernel,
        out_shape=(
            jax.ShapeDtypeStruct(agg.shape, agg.dtype),
            jax.ShapeDtypeStruct(res.shape, res.dtype),
        ),
    )(agg, res)


def kernel(user_emb, entity_emb, edge_index, edge_type, inter_edge, inter_edge_w, relation_emb, W_Q):
    head = edge_index[0]
    tail = edge_index[1]
    u_idx = inter_edge[0]
    i_idx = inter_edge[1]
    ent = entity_emb
    ent_res = entity_emb
    usr_res = user_emb
    for _ in range(2):
        query = (ent[head] @ W_Q).reshape(-1, N_HEADS, D_K)
        key = (ent[tail] @ W_Q).reshape(-1, N_HEADS, D_K)
        rel = relation_emb[edge_type - 1]
        key = key * rel.reshape(-1, N_HEADS, D_K)
        edge_attn_score = (query * key).sum(axis=-1) / math.sqrt(D_K)
        neigh_relation_emb = ent[tail] * rel
        value = neigh_relation_emb.reshape(-1, N_HEADS, D_K)
        attn = _scatter_softmax(edge_attn_score, head, N_ENT)
        entity_agg = (value * attn[:, :, None]).reshape(-1, N_HEADS * D_K)
        entity_agg = jax.ops.segment_sum(entity_agg, head, num_segments=N_ENT)
        item_agg = inter_edge_w[:, None] * ent[i_idx]
        user_agg = jax.ops.segment_sum(item_agg, u_idx, num_segments=N_USR)
        ent, ent_res = _norm_res(entity_agg, ent_res)
        _, usr_res = _norm_res(user_agg, usr_res)
    return (ent_res, usr_res)


# speedup vs baseline: 1.5762x; 1.5569x over previous
import functools

import jax
import jax.numpy as jnp
from jax import lax
from jax.experimental import pallas as pl
from jax.experimental.pallas import tpu as pltpu
from jax.experimental.pallas import tpu_sc as plsc

N = 10000
E = 320000
CH = 128
NREL = 7
NC = 2
NS = 16
NW = NC * NS
EPW = E // NW
B = 80
NCH = EPW // B
RB = 1000
NEG = -3.0e38

_MESH = plsc.VectorSubcoreMesh(core_axis_name="c", subcore_axis_name="s")
_CP = pltpu.CompilerParams(needs_layout_passes=False)


def _wid():
    return lax.axis_index("s") * NC + lax.axis_index("c")


def _prep_body(ent_ref, wq_ref, rel_ref, t_ref, kr_ref, er_ref):
    r = pl.program_id(1)
    tblk = jnp.dot(ent_ref[...], wq_ref[...], preferred_element_type=jnp.float32)
    relrow = rel_ref[pl.ds(r, 1), :]

    @pl.when(r == 0)
    def _():
        t_ref[...] = tblk

    kr_ref[...] = (tblk * relrow)[None]
    er_ref[...] = (ent_ref[...] * relrow)[None]


def _tc_prep(ent, wq, rel):
    return pl.pallas_call(
        _prep_body,
        grid=(N // RB, NREL),
        in_specs=[
            pl.BlockSpec((RB, CH), lambda j, r: (j, 0)),
            pl.BlockSpec((CH, CH), lambda j, r: (0, 0)),
            pl.BlockSpec((NREL, CH), lambda j, r: (0, 0)),
        ],
        out_specs=[
            pl.BlockSpec((RB, CH), lambda j, r: (j, 0)),
            pl.BlockSpec((1, RB, CH), lambda j, r: (r, j, 0)),
            pl.BlockSpec((1, RB, CH), lambda j, r: (r, j, 0)),
        ],
        out_shape=[
            jax.ShapeDtypeStruct((N, CH), jnp.float32),
            jax.ShapeDtypeStruct((NREL, N, CH), jnp.float32),
            jax.ShapeDtypeStruct((NREL, N, CH), jnp.float32),
        ],
    )(ent, wq, rel)


def _gmax_body(mx_ref, out_ref):
    out_ref[...] = jnp.max(mx_ref[...], axis=0)


def _tc_gmax(maxtab):
    return pl.pallas_call(
        _gmax_body,
        out_shape=jax.ShapeDtypeStruct((2 * N,), jnp.float32),
    )(maxtab)


def _lden_body(gm_ref, sm_ref, out_ref):
    den = jnp.sum(sm_ref[...], axis=0) + 1e-16
    out_ref[...] = gm_ref[...] + jnp.log(den)


def _tc_lden(gmax, sumtab):
    return pl.pallas_call(
        _lden_body,
        out_shape=jax.ShapeDtypeStruct((2 * N,), jnp.float32),
    )(gmax, sumtab)


def _merge_body(p_ref, res_ref, xn_ref, rn_ref):
    x = p_ref[0] + p_ref[1]
    n = jnp.sqrt(jnp.sum(x * x, axis=1, keepdims=True))
    xn = x / jnp.maximum(n, 1e-12)
    xn_ref[...] = xn
    rn_ref[...] = res_ref[...] + xn


def _tc_merge(p, res):
    return pl.pallas_call(
        _merge_body,
        grid=(N // RB,),
        in_specs=[
            pl.BlockSpec((2, RB, CH), lambda j: (0, j, 0)),
            pl.BlockSpec((RB, CH), lambda j: (j, 0)),
        ],
        out_specs=[
            pl.BlockSpec((RB, CH), lambda j: (j, 0)),
            pl.BlockSpec((RB, CH), lambda j: (j, 0)),
        ],
        out_shape=[
            jax.ShapeDtypeStruct((N, CH), jnp.float32),
            jax.ShapeDtypeStruct((N, CH), jnp.float32),
        ],
    )(p, res)


def _lanes_i32():
    return lax.broadcasted_iota(jnp.int32, (16,), 0)


def _tournament_max(tab, hvv, vals):
    def cond(pend):
        return jnp.any(pend)

    def body(pend):
        cur = plsc.load_gather(tab, [hvv])
        need = jnp.logical_and(vals > cur, pend)
        plsc.store_scatter(tab, [hvv], vals, mask=need)
        cur2 = plsc.load_gather(tab, [hvv])
        return vals > cur2

    lax.while_loop(cond, body, jnp.full((16,), True, dtype=jnp.bool_))


def _tournament_add2(sm, tag, hvv, v0, v1):
    lid = _lanes_i32()

    def cond(pend):
        return jnp.any(pend)

    def body(pend):
        plsc.store_scatter(tag, [hvv], lid, mask=pend)
        got = plsc.load_gather(tag, [hvv])
        win = jnp.logical_and(got == lid, pend)
        c0 = plsc.load_gather(sm, [hvv])
        plsc.store_scatter(sm, [hvv], c0 + v0, mask=win)
        c1 = plsc.load_gather(sm, [hvv + N])
        plsc.store_scatter(sm, [hvv + N], c1 + v1, mask=win)
        return jnp.logical_and(pend, jnp.logical_not(win))

    lax.while_loop(cond, body, jnp.full((16,), True, dtype=jnp.bool_))


def _fill_flat(ref, n, val):
    v = jnp.full((16,), val, jnp.float32)

    def body(c, _):
        ref[pl.ds(c * 16, 16)] = v
        return 0

    lax.fori_loop(0, n // 16, body, 0)


@functools.partial(
    pl.kernel,
    out_type=(
        jax.ShapeDtypeStruct((NW, 2 * EPW), jnp.float32),
        jax.ShapeDtypeStruct((NW, 2 * N), jnp.float32),
    ),
    mesh=_MESH,
    compiler_params=_CP,
    scratch_types=[
        pltpu.VMEM((NCH, B), jnp.int32),
        pltpu.VMEM((NCH, B), jnp.int32),
        pltpu.VMEM((B, CH), jnp.float32),
        pltpu.VMEM((B, CH), jnp.float32),
        pltpu.VMEM((B, CH), jnp.float32),
        pltpu.VMEM((B, CH), jnp.float32),
        pltpu.VMEM((2 * EPW,), jnp.float32),
        pltpu.VMEM((2 * N,), jnp.float32),
        pltpu.SemaphoreType.DMA,
    ],
)
def _sc_scores(t_hbm, kr_hbm, head_hbm, kt_hbm, scores_hbm, maxtab_hbm,
               hv_all, kv_all, qr0, qr1, kr0, kr1, sb, mx, sem_g):
    wid = _wid()

    pltpu.sync_copy(head_hbm.at[wid], hv_all)
    pltpu.sync_copy(kt_hbm.at[wid], kv_all)
    _fill_flat(mx, 2 * N, NEG)

    def compute(jc, qv2, kv2):
        pltpu.make_async_copy(t_hbm.at[hv_all.at[jc]], qv2, sem_g).wait()
        pltpu.make_async_copy(kr_hbm.at[kv_all.at[jc]], kv2, sem_g).wait()

        def group(g, _):
            base = g * 16
            el = _lanes_i32() + base

            def half(lo):
                acc = jnp.zeros((16,), jnp.float32)
                for ch in range(lo, lo + 64):
                    cv = jnp.full((16,), ch, jnp.int32)
                    acc = acc + (plsc.load_gather(qv2, [el, cv])
                                 * plsc.load_gather(kv2, [el, cv]))
                return acc * 0.125

            s0 = half(0)
            s1 = half(64)
            sb[pl.ds(jc * B + base, 16)] = s0
            sb[pl.ds(EPW + jc * B + base, 16)] = s1
            hvv = hv_all[jc, pl.ds(base, 16)]
            _tournament_max(mx, hvv, s0)
            _tournament_max(mx, hvv + N, s1)
            return 0

        lax.fori_loop(0, B // 16, group, 0)

    def loop(j, _):
        even_j = lax.rem(j, 2) == 0

        @pl.when(jnp.logical_and(j < NCH, even_j))
        def _():
            pltpu.async_copy(t_hbm.at[hv_all.at[j]], qr0, sem_g)
            pltpu.async_copy(kr_hbm.at[kv_all.at[j]], kr0, sem_g)

        @pl.when(jnp.logical_and(j < NCH, jnp.logical_not(even_j)))
        def _():
            pltpu.async_copy(t_hbm.at[hv_all.at[j]], qr1, sem_g)
            pltpu.async_copy(kr_hbm.at[kv_all.at[j]], kr1, sem_g)

        @pl.when(jnp.logical_and(j >= 1, jnp.logical_not(even_j)))
        def _():
            compute(j - 1, qr0, kr0)

        @pl.when(jnp.logical_and(j >= 1, even_j))
        def _():
            compute(j - 1, qr1, kr1)

        return 0

    lax.fori_loop(0, NCH + 1, loop, 0)
    pltpu.sync_copy(sb, scores_hbm.at[wid])
    pltpu.sync_copy(mx, maxtab_hbm.at[wid])


@functools.partial(
    pl.kernel,
    out_type=jax.ShapeDtypeStruct((NW, 2 * N), jnp.float32),
    mesh=_MESH,
    compiler_params=_CP,
    scratch_types=[
        pltpu.VMEM((2 * EPW,), jnp.float32),
        pltpu.VMEM((NCH, B), jnp.int32),
        pltpu.VMEM((2 * N,), jnp.float32),
        pltpu.VMEM((2 * N,), jnp.float32),
        pltpu.VMEM((N,), jnp.int32),
    ],
)
def _sc_ssum(scores_hbm, head_hbm, gmax_hbm, sumtab_hbm,
             sb, hv_all, gm, sm, tag):
    wid = _wid()

    pltpu.sync_copy(scores_hbm.at[wid], sb)
    pltpu.sync_copy(head_hbm.at[wid], hv_all)
    pltpu.sync_copy(gmax_hbm, gm)
    _fill_flat(sm, 2 * N, 0.0)

    def loop(j, _):
        def group(g, _):
            base = g * 16
            hvv = hv_all[j, pl.ds(base, 16)]
            s0 = sb[pl.ds(j * B + base, 16)]
            s1 = sb[pl.ds(EPW + j * B + base, 16)]
            e0 = jnp.exp(s0 - plsc.load_gather(gm, [hvv]))
            e1 = jnp.exp(s1 - plsc.load_gather(gm, [hvv + N]))
            _tournament_add2(sm, tag, hvv, e0, e1)
            return 0

        lax.fori_loop(0, B // 16, group, 0)
        return 0

    lax.fori_loop(0, NCH, loop, 0)
    pltpu.sync_copy(sm, sumtab_hbm.at[wid])


@functools.partial(
    pl.kernel,
    out_type=jax.ShapeDtypeStruct((NW, NCH, 2 * B), jnp.float32),
    mesh=_MESH,
    compiler_params=_CP,
    scratch_types=[
        pltpu.VMEM((2 * EPW,), jnp.float32),
        pltpu.VMEM((NCH, B), jnp.int32),
        pltpu.VMEM((2 * N,), jnp.float32),
        pltpu.VMEM((NCH, 2 * B), jnp.float32),
    ],
)
def _sc_attn(scores_hbm, head_hbm, lden_hbm, attn_hbm, sb, hv_all, ld, ab):
    wid = _wid()

    pltpu.sync_copy(scores_hbm.at[wid], sb)
    pltpu.sync_copy(head_hbm.at[wid], hv_all)
    pltpu.sync_copy(lden_hbm, ld)

    def loop(j, _):
        def group(g, _):
            base = g * 16
            hvv = hv_all[j, pl.ds(base, 16)]
            s0 = sb[pl.ds(j * B + base, 16)]
            s1 = sb[pl.ds(EPW + j * B + base, 16)]
            ab[j, pl.ds(base, 16)] = jnp.exp(s0 - plsc.load_gather(ld, [hvv]))
            ab[j, pl.ds(B + base, 16)] = jnp.exp(
                s1 - plsc.load_gather(ld, [hvv + N]))
            return 0

        lax.fori_loop(0, B // 16, group, 0)
        return 0

    lax.fori_loop(0, NCH, loop, 0)
    pltpu.sync_copy(ab, attn_hbm.at[wid])


@functools.partial(
    pl.kernel,
    out_type=(
        jax.ShapeDtypeStruct((2, N, CH), jnp.float32),
        jax.ShapeDtypeStruct((2, N, CH), jnp.float32),
    ),
    mesh=_MESH,
    compiler_params=_CP,
    scratch_types=[
        pltpu.VMEM((B,), jnp.int32),
        pltpu.VMEM((B,), jnp.int32),
        pltpu.VMEM((B,), jnp.int32),
        pltpu.VMEM((B,), jnp.int32),
        pltpu.VMEM((2 * B,), jnp.float32),
        pltpu.VMEM((2 * B,), jnp.float32),
        pltpu.VMEM((B,), jnp.float32),
        pltpu.VMEM((B,), jnp.float32),
        pltpu.VMEM((B,), jnp.int32),
        pltpu.VMEM((B,), jnp.int32),
        pltpu.VMEM((B, CH), jnp.float32),
        pltpu.VMEM((B, CH), jnp.float32),
        pltpu.VMEM((B, CH), jnp.float32),
        pltpu.VMEM((B, CH), jnp.float32),
        pltpu.VMEM((8, CH), jnp.float32),
        pltpu.VMEM_SHARED((N, CH), jnp.float32),
        pltpu.SemaphoreType.DMA,
        pltpu.SemaphoreType.DMA,
        pltpu.SemaphoreType.DMA,
    ],
)
def _sc_agg(er_hbm, kt_hbm, head_hbm, attn_hbm, ent_hbm, i_hbm, u_hbm, w_hbm,
            pent_hbm, pusr_hbm,
            hr0, hr1, sr0, sr1, ar0, ar1, wr0, wr1, hs0, hs1,
            vr0, vr1, ob0, ob1, zb, agg, sem_i, sem_g, sem_s):
    wid = _wid()
    cid = lax.axis_index("c")
    sid = lax.axis_index("s")
    off = jnp.minimum(sid * 640, N - 640)

    zv = jnp.zeros((16,), jnp.float32)
    for r in range(8):
        for c in range(8):
            zb[r, pl.ds(c * 16, 16)] = zv

    def zero_agg():
        def zloop(i, _):
            pltpu.sync_copy(zb, agg.at[pl.ds(off + i * 8, 8), :])
            return 0

        lax.fori_loop(0, 80, zloop, 0)

    def run_phase(row_tab, dst_hbm, src_hbm, a_hbm, a0r, a1r, user_mode):
        def fire_idx(j, hr, sr, ar):
            pltpu.async_copy(dst_hbm.at[wid, j], hr, sem_i)
            pltpu.async_copy(src_hbm.at[wid, j], sr, sem_i)
            pltpu.async_copy(a_hbm.at[wid, j], ar, sem_i)

        def wait_idx(j, hr, sr, ar):
            pltpu.make_async_copy(dst_hbm.at[wid, j], hr, sem_i).wait()
            pltpu.make_async_copy(src_hbm.at[wid, j], sr, sem_i).wait()
            pltpu.make_async_copy(a_hbm.at[wid, j], ar, sem_i).wait()

        def compute(jc, j, sr, vv2, ob2, hr, ar, hs):
            pltpu.make_async_copy(row_tab.at[sr], vv2, sem_g).wait()

            @pl.when(j >= 3)
            def _():
                pltpu.make_async_copy(ob2, agg.at[hs], sem_s).wait()

            def hcopy(c, _):
                hs[pl.ds(c * 16, 16)] = hr[pl.ds(c * 16, 16)]
                return 0

            lax.fori_loop(0, B // 16, hcopy, 0)

            def group(g, _):
                base = g * 16
                a0 = ar[pl.ds(base, 16)]
                if user_mode:
                    a1 = a0
                else:
                    a1 = ar[pl.ds(B + base, 16)]
                el = _lanes_i32() + base
                for ch in range(CH):
                    cv = jnp.full((16,), ch, jnp.int32)
                    a = a0 if ch < 64 else a1
                    plsc.store_scatter(ob2, [el, cv],
                                       plsc.load_gather(vv2, [el, cv]) * a)
                return 0

            lax.fori_loop(0, B // 16, group, 0)
            pltpu.async_copy(ob2, agg.at[hs], sem_s, add=True)

        fire_idx(0, hr0, sr0, a0r)

        def loop(j, _):
            even_j = lax.rem(j, 2) == 0

            @pl.when(jnp.logical_and(j < NCH, even_j))
            def _():
                wait_idx(j, hr0, sr0, a0r)
                pltpu.async_copy(row_tab.at[sr0], vr0, sem_g)

            @pl.when(jnp.logical_and(j < NCH, jnp.logical_not(even_j)))
            def _():
                wait_idx(j, hr1, sr1, a1r)
                pltpu.async_copy(row_tab.at[sr1], vr1, sem_g)

            @pl.when(jnp.logical_and(j >= 1, jnp.logical_not(even_j)))
            def _():
                compute(j - 1, j, sr0, vr0, ob0, hr0, a0r, hs0)

            @pl.when(jnp.logical_and(j >= 1, even_j))
            def _():
                compute(j - 1, j, sr1, vr1, ob1, hr1, a1r, hs1)

            @pl.when(j + 1 < NCH)
            def _():
                even_n = lax.rem(j + 1, 2) == 0

                @pl.when(even_n)
                def _():
                    fire_idx(j + 1, hr0, sr0, a0r)

                @pl.when(jnp.logical_not(even_n))
                def _():
                    fire_idx(j + 1, hr1, sr1, a1r)

            return 0

        lax.fori_loop(0, NCH + 1, loop, 0)
        pltpu.make_async_copy(ob0, agg.at[hs0], sem_s).wait()
        pltpu.make_async_copy(ob1, agg.at[hs1], sem_s).wait()

    zero_agg()
    plsc.subcore_barrier()
    run_phase(er_hbm, head_hbm, kt_hbm, attn_hbm, ar0, ar1, False)
    plsc.subcore_barrier()
    pltpu.sync_copy(agg.at[pl.ds(off, 640), :],
                    pent_hbm.at[cid, pl.ds(off, 640), :])
    plsc.subcore_barrier()

    zero_agg()
    plsc.subcore_barrier()
    run_phase(ent_hbm, u_hbm, i_hbm, w_hbm, wr0, wr1, True)
    plsc.subcore_barrier()
    pltpu.sync_copy(agg.at[pl.ds(off, 640), :],
                    pusr_hbm.at[cid, pl.ds(off, 640), :])


def kernel(user_emb, entity_emb, edge_index, edge_type, inter_edge, inter_edge_w,
           relation_emb, W_Q):
    head2 = edge_index[0].reshape(NW, NCH, B)
    ktidx = ((edge_type - 1) % NREL) * N + edge_index[1]
    kt2 = ktidx.reshape(NW, NCH, B)
    u2 = inter_edge[0].reshape(NW, NCH, B)
    i2 = inter_edge[1].reshape(NW, NCH, B)
    w2 = inter_edge_w.reshape(NW, NCH, B)

    ent = entity_emb
    ent_res = entity_emb
    usr_res = user_emb
    for _ in range(2):
        T, KR3, ER3 = _tc_prep(ent, W_Q, relation_emb)
        KR = KR3.reshape(NREL * N, CH)
        ER = ER3.reshape(NREL * N, CH)
        scores, maxtab = _sc_scores(T, KR, head2, kt2)
        gmax = _tc_gmax(maxtab)
        sumtab = _sc_ssum(scores, head2, gmax)
        lden = _tc_lden(gmax, sumtab)
        attn = _sc_attn(scores, head2, lden)
        pent, pusr = _sc_agg(ER, kt2, head2, attn, ent, i2, u2, w2)
        ent, ent_res = _tc_merge(pent, ent_res)
        _, usr_res = _tc_merge(pusr, usr_res)
    return (ent_res, usr_res)


# speedup vs baseline: 8.2772x; 5.2514x over previous
import functools

import jax
import jax.numpy as jnp
from jax import lax
from jax.experimental import pallas as pl
from jax.experimental.pallas import tpu as pltpu
from jax.experimental.pallas import tpu_sc as plsc

N = 10000
E = 320000
CH = 128
NREL = 7
NC = 2
NS = 16
NW = NC * NS
EPW = E // NW
B = 80
NCH = EPW // B
RB = 1000
NEG = -3.0e38

_MESH = plsc.VectorSubcoreMesh(core_axis_name="c", subcore_axis_name="s")
_CP = pltpu.CompilerParams(needs_layout_passes=False)


def _wid():
    return lax.axis_index("s") * NC + lax.axis_index("c")


def _prep_body(ent_ref, wq_ref, rel_ref, t_ref, kr_ref, er_ref):
    r = pl.program_id(1)
    tblk = jnp.dot(ent_ref[...], wq_ref[...], preferred_element_type=jnp.float32)
    relrow = rel_ref[pl.ds(r, 1), :]

    @pl.when(r == 0)
    def _():
        t_ref[...] = tblk

    kr_ref[...] = (tblk * relrow)[None]
    er_ref[...] = (ent_ref[...] * relrow)[None]


def _tc_prep(ent, wq, rel):
    return pl.pallas_call(
        _prep_body,
        grid=(N // RB, NREL),
        in_specs=[
            pl.BlockSpec((RB, CH), lambda j, r: (j, 0)),
            pl.BlockSpec((CH, CH), lambda j, r: (0, 0)),
            pl.BlockSpec((NREL, CH), lambda j, r: (0, 0)),
        ],
        out_specs=[
            pl.BlockSpec((RB, CH), lambda j, r: (j, 0)),
            pl.BlockSpec((1, RB, CH), lambda j, r: (r, j, 0)),
            pl.BlockSpec((1, RB, CH), lambda j, r: (r, j, 0)),
        ],
        out_shape=[
            jax.ShapeDtypeStruct((N, CH), jnp.float32),
            jax.ShapeDtypeStruct((NREL, N, CH), jnp.float32),
            jax.ShapeDtypeStruct((NREL, N, CH), jnp.float32),
        ],
    )(ent, wq, rel)


def _gmax_body(mx_ref, out_ref):
    out_ref[...] = jnp.max(mx_ref[...], axis=0)


def _tc_gmax(maxtab):
    return pl.pallas_call(
        _gmax_body,
        out_shape=jax.ShapeDtypeStruct((2 * N,), jnp.float32),
    )(maxtab)


def _lden_body(gm_ref, sm_ref, out_ref):
    den = jnp.sum(sm_ref[...], axis=0) + 1e-16
    out_ref[...] = gm_ref[...] + jnp.log(den)


def _tc_lden(gmax, sumtab):
    return pl.pallas_call(
        _lden_body,
        out_shape=jax.ShapeDtypeStruct((2 * N,), jnp.float32),
    )(gmax, sumtab)


def _merge_body(p_ref, res_ref, xn_ref, rn_ref):
    x = p_ref[0] + p_ref[1]
    n = jnp.sqrt(jnp.sum(x * x, axis=1, keepdims=True))
    xn = x / jnp.maximum(n, 1e-12)
    xn_ref[...] = xn
    rn_ref[...] = res_ref[...] + xn


def _tc_merge(p, res):
    return pl.pallas_call(
        _merge_body,
        grid=(N // RB,),
        in_specs=[
            pl.BlockSpec((2, RB, CH), lambda j: (0, j, 0)),
            pl.BlockSpec((RB, CH), lambda j: (j, 0)),
        ],
        out_specs=[
            pl.BlockSpec((RB, CH), lambda j: (j, 0)),
            pl.BlockSpec((RB, CH), lambda j: (j, 0)),
        ],
        out_shape=[
            jax.ShapeDtypeStruct((N, CH), jnp.float32),
            jax.ShapeDtypeStruct((N, CH), jnp.float32),
        ],
    )(p, res)


def _lanes_i32():
    return lax.broadcasted_iota(jnp.int32, (16,), 0)


_GDN = lax.GatherDimensionNumbers(
    offset_dims=(), collapsed_slice_dims=(0,), start_index_map=(0,))


def _perm(v, idx):
    return lax.gather(v, idx[:, None], _GDN, slice_sizes=(1,),
                      mode=lax.GatherScatterMode.PROMISE_IN_BOUNDS)


def _bsum(v):
    return _perm(plsc.cumsum(v), jnp.full((16,), 15, jnp.int32))


def _bcast(v, e):
    return _perm(v, jnp.full((16,), e, jnp.int32))


def _tournament_max2(mx, hvv, s0, s1):
    hv1 = hvv + N

    def cond(pend):
        return jnp.any(pend)

    def body(pend):
        c0 = plsc.load_gather(mx, [hvv])
        c1 = plsc.load_gather(mx, [hv1])
        plsc.store_scatter(mx, [hvv], s0, mask=jnp.logical_and(s0 > c0, pend))
        plsc.store_scatter(mx, [hv1], s1, mask=jnp.logical_and(s1 > c1, pend))
        d0 = plsc.load_gather(mx, [hvv])
        d1 = plsc.load_gather(mx, [hv1])
        return jnp.logical_or(s0 > d0, s1 > d1)

    lax.while_loop(cond, body, jnp.full((16,), True, dtype=jnp.bool_))


def _tournament_add2(sm, tag, hvv, v0, v1):
    lid = _lanes_i32()

    def cond(pend):
        return jnp.any(pend)

    def body(pend):
        plsc.store_scatter(tag, [hvv], lid, mask=pend)
        got = plsc.load_gather(tag, [hvv])
        win = jnp.logical_and(got == lid, pend)
        c0 = plsc.load_gather(sm, [hvv])
        plsc.store_scatter(sm, [hvv], c0 + v0, mask=win)
        c1 = plsc.load_gather(sm, [hvv + N])
        plsc.store_scatter(sm, [hvv + N], c1 + v1, mask=win)
        return jnp.logical_and(pend, jnp.logical_not(win))

    lax.while_loop(cond, body, jnp.full((16,), True, dtype=jnp.bool_))


def _fill_flat(ref, n, val):
    v = jnp.full((16,), val, jnp.float32)

    def body(c, _):
        ref[pl.ds(c * 16, 16)] = v
        return 0

    lax.fori_loop(0, n // 16, body, 0)


@functools.partial(
    pl.kernel,
    out_type=(
        jax.ShapeDtypeStruct((NW, 2 * EPW), jnp.float32),
        jax.ShapeDtypeStruct((NW, 2 * N), jnp.float32),
    ),
    mesh=_MESH,
    compiler_params=_CP,
    scratch_types=[
        pltpu.VMEM((NCH, B), jnp.int32),
        pltpu.VMEM((NCH, B), jnp.int32),
        pltpu.VMEM((B, CH), jnp.float32),
        pltpu.VMEM((B, CH), jnp.float32),
        pltpu.VMEM((B, CH), jnp.float32),
        pltpu.VMEM((B, CH), jnp.float32),
        pltpu.VMEM((2 * EPW,), jnp.float32),
        pltpu.VMEM((2 * N,), jnp.float32),
        pltpu.SemaphoreType.DMA,
    ],
)
def _sc_scores(t_hbm, kr_hbm, head_hbm, kt_hbm, scores_hbm, maxtab_hbm,
               hv_all, kv_all, qr0, qr1, kr0, kr1, sb, mx, sem_g):
    wid = _wid()

    pltpu.sync_copy(head_hbm.at[wid], hv_all)
    pltpu.sync_copy(kt_hbm.at[wid], kv_all)
    _fill_flat(mx, 2 * N, NEG)

    def compute(jc, qv2, kv2):
        pltpu.make_async_copy(t_hbm.at[hv_all.at[jc]], qv2, sem_g).wait()
        pltpu.make_async_copy(kr_hbm.at[kv_all.at[jc]], kv2, sem_g).wait()

        lid = _lanes_i32()

        def group(g, _):
            base = g * 16
            s0 = jnp.zeros((16,), jnp.float32)
            s1 = jnp.zeros((16,), jnp.float32)
            for e in range(16):
                ea = base + e

                def half(lo):
                    acc = (qv2[ea, pl.ds(lo, 16)] * kv2[ea, pl.ds(lo, 16)])
                    for t in range(1, 4):
                        acc = acc + (qv2[ea, pl.ds(lo + t * 16, 16)]
                                     * kv2[ea, pl.ds(lo + t * 16, 16)])
                    return _bsum(acc)

                msk = lid == e
                s0 = jnp.where(msk, half(0), s0)
                s1 = jnp.where(msk, half(64), s1)
            s0 = s0 * 0.125
            s1 = s1 * 0.125
            sb[pl.ds(jc * B + base, 16)] = s0
            sb[pl.ds(EPW + jc * B + base, 16)] = s1
            hvv = hv_all[jc, pl.ds(base, 16)]
            _tournament_max2(mx, hvv, s0, s1)
            return 0

        lax.fori_loop(0, B // 16, group, 0)

    def loop(j, _):
        even_j = lax.rem(j, 2) == 0

        @pl.when(jnp.logical_and(j < NCH, even_j))
        def _():
            pltpu.async_copy(t_hbm.at[hv_all.at[j]], qr0, sem_g)
            pltpu.async_copy(kr_hbm.at[kv_all.at[j]], kr0, sem_g)

        @pl.when(jnp.logical_and(j < NCH, jnp.logical_not(even_j)))
        def _():
            pltpu.async_copy(t_hbm.at[hv_all.at[j]], qr1, sem_g)
            pltpu.async_copy(kr_hbm.at[kv_all.at[j]], kr1, sem_g)

        @pl.when(jnp.logical_and(j >= 1, jnp.logical_not(even_j)))
        def _():
            compute(j - 1, qr0, kr0)

        @pl.when(jnp.logical_and(j >= 1, even_j))
        def _():
            compute(j - 1, qr1, kr1)

        return 0

    lax.fori_loop(0, NCH + 1, loop, 0)
    pltpu.sync_copy(sb, scores_hbm.at[wid])
    pltpu.sync_copy(mx, maxtab_hbm.at[wid])


@functools.partial(
    pl.kernel,
    out_type=jax.ShapeDtypeStruct((NW, 2 * N), jnp.float32),
    mesh=_MESH,
    compiler_params=_CP,
    scratch_types=[
        pltpu.VMEM((2 * EPW,), jnp.float32),
        pltpu.VMEM((NCH, B), jnp.int32),
        pltpu.VMEM((2 * N,), jnp.float32),
        pltpu.VMEM((2 * N,), jnp.float32),
        pltpu.VMEM((N,), jnp.int32),
    ],
)
def _sc_ssum(scores_hbm, head_hbm, gmax_hbm, sumtab_hbm,
             sb, hv_all, gm, sm, tag):
    wid = _wid()

    pltpu.sync_copy(scores_hbm.at[wid], sb)
    pltpu.sync_copy(head_hbm.at[wid], hv_all)
    pltpu.sync_copy(gmax_hbm, gm)
    _fill_flat(sm, 2 * N, 0.0)

    def loop(j, _):
        def group(g, _):
            base = g * 16
            hvv = hv_all[j, pl.ds(base, 16)]
            s0 = sb[pl.ds(j * B + base, 16)]
            s1 = sb[pl.ds(EPW + j * B + base, 16)]
            e0 = jnp.exp(s0 - plsc.load_gather(gm, [hvv]))
            e1 = jnp.exp(s1 - plsc.load_gather(gm, [hvv + N]))
            _tournament_add2(sm, tag, hvv, e0, e1)
            return 0

        lax.fori_loop(0, B // 16, group, 0)
        return 0

    lax.fori_loop(0, NCH, loop, 0)
    pltpu.sync_copy(sm, sumtab_hbm.at[wid])


@functools.partial(
    pl.kernel,
    out_type=jax.ShapeDtypeStruct((NW, NCH, 2 * B), jnp.float32),
    mesh=_MESH,
    compiler_params=_CP,
    scratch_types=[
        pltpu.VMEM((2 * EPW,), jnp.float32),
        pltpu.VMEM((NCH, B), jnp.int32),
        pltpu.VMEM((2 * N,), jnp.float32),
        pltpu.VMEM((NCH, 2 * B), jnp.float32),
    ],
)
def _sc_attn(scores_hbm, head_hbm, lden_hbm, attn_hbm, sb, hv_all, ld, ab):
    wid = _wid()

    pltpu.sync_copy(scores_hbm.at[wid], sb)
    pltpu.sync_copy(head_hbm.at[wid], hv_all)
    pltpu.sync_copy(lden_hbm, ld)

    def loop(j, _):
        def group(g, _):
            base = g * 16
            hvv = hv_all[j, pl.ds(base, 16)]
            s0 = sb[pl.ds(j * B + base, 16)]
            s1 = sb[pl.ds(EPW + j * B + base, 16)]
            ab[j, pl.ds(base, 16)] = jnp.exp(s0 - plsc.load_gather(ld, [hvv]))
            ab[j, pl.ds(B + base, 16)] = jnp.exp(
                s1 - plsc.load_gather(ld, [hvv + N]))
            return 0

        lax.fori_loop(0, B // 16, group, 0)
        return 0

    lax.fori_loop(0, NCH, loop, 0)
    pltpu.sync_copy(ab, attn_hbm.at[wid])


@functools.partial(
    pl.kernel,
    out_type=(
        jax.ShapeDtypeStruct((2, N, CH), jnp.float32),
        jax.ShapeDtypeStruct((2, N, CH), jnp.float32),
    ),
    mesh=_MESH,
    compiler_params=_CP,
    scratch_types=[
        pltpu.VMEM((B,), jnp.int32),
        pltpu.VMEM((B,), jnp.int32),
        pltpu.VMEM((B,), jnp.int32),
        pltpu.VMEM((B,), jnp.int32),
        pltpu.VMEM((2 * B,), jnp.float32),
        pltpu.VMEM((2 * B,), jnp.float32),
        pltpu.VMEM((B,), jnp.float32),
        pltpu.VMEM((B,), jnp.float32),
        pltpu.VMEM((B,), jnp.int32),
        pltpu.VMEM((B,), jnp.int32),
        pltpu.VMEM((B, CH), jnp.float32),
        pltpu.VMEM((B, CH), jnp.float32),
        pltpu.VMEM((B, CH), jnp.float32),
        pltpu.VMEM((B, CH), jnp.float32),
        pltpu.VMEM((8, CH), jnp.float32),
        pltpu.VMEM_SHARED((N, CH), jnp.float32),
        pltpu.SemaphoreType.DMA,
        pltpu.SemaphoreType.DMA,
        pltpu.SemaphoreType.DMA,
    ],
)
def _sc_agg(er_hbm, kt_hbm, head_hbm, attn_hbm, ent_hbm, i_hbm, u_hbm, w_hbm,
            pent_hbm, pusr_hbm,
            hr0, hr1, sr0, sr1, ar0, ar1, wr0, wr1, hs0, hs1,
            vr0, vr1, ob0, ob1, zb, agg, sem_i, sem_g, sem_s):
    wid = _wid()
    cid = lax.axis_index("c")
    sid = lax.axis_index("s")
    off = jnp.minimum(sid * 640, N - 640)

    zv = jnp.zeros((16,), jnp.float32)
    for r in range(8):
        for c in range(8):
            zb[r, pl.ds(c * 16, 16)] = zv

    def zero_agg():
        def zloop(i, _):
            pltpu.sync_copy(zb, agg.at[pl.ds(off + i * 8, 8), :])
            return 0

        lax.fori_loop(0, 80, zloop, 0)

    def run_phase(row_tab, dst_hbm, src_hbm, a_hbm, a0r, a1r, user_mode):
        def fire_idx(j, hr, sr, ar):
            pltpu.async_copy(dst_hbm.at[wid, j], hr, sem_i)
            pltpu.async_copy(src_hbm.at[wid, j], sr, sem_i)
            pltpu.async_copy(a_hbm.at[wid, j], ar, sem_i)

        def wait_idx(j, hr, sr, ar):
            pltpu.make_async_copy(dst_hbm.at[wid, j], hr, sem_i).wait()
            pltpu.make_async_copy(src_hbm.at[wid, j], sr, sem_i).wait()
            pltpu.make_async_copy(a_hbm.at[wid, j], ar, sem_i).wait()

        def compute(jc, j, sr, vv2, ob2, hr, ar, hs):
            pltpu.make_async_copy(row_tab.at[sr], vv2, sem_g).wait()

            @pl.when(j >= 3)
            def _():
                pltpu.make_async_copy(ob2, agg.at[hs], sem_s).wait()

            def hcopy(c, _):
                hs[pl.ds(c * 16, 16)] = hr[pl.ds(c * 16, 16)]
                return 0

            lax.fori_loop(0, B // 16, hcopy, 0)

            def group(g, _):
                base = g * 16
                a0 = ar[pl.ds(base, 16)]
                if user_mode:
                    a1 = a0
                else:
                    a1 = ar[pl.ds(B + base, 16)]
                for e in range(16):
                    ea = base + e
                    bc0 = _bcast(a0, e)
                    bc1 = bc0 if user_mode else _bcast(a1, e)
                    for k in range(8):
                        bc = bc0 if k < 4 else bc1
                        ob2[ea, pl.ds(k * 16, 16)] = (
                            vv2[ea, pl.ds(k * 16, 16)] * bc)
                return 0

            lax.fori_loop(0, B // 16, group, 0)
            pltpu.async_copy(ob2, agg.at[hs], sem_s, add=True)

        fire_idx(0, hr0, sr0, a0r)

        def loop(j, _):
            even_j = lax.rem(j, 2) == 0

            @pl.when(jnp.logical_and(j < NCH, even_j))
            def _():
                wait_idx(j, hr0, sr0, a0r)
                pltpu.async_copy(row_tab.at[sr0], vr0, sem_g)

            @pl.when(jnp.logical_and(j < NCH, jnp.logical_not(even_j)))
            def _():
                wait_idx(j, hr1, sr1, a1r)
                pltpu.async_copy(row_tab.at[sr1], vr1, sem_g)

            @pl.when(jnp.logical_and(j >= 1, jnp.logical_not(even_j)))
            def _():
                compute(j - 1, j, sr0, vr0, ob0, hr0, a0r, hs0)

            @pl.when(jnp.logical_and(j >= 1, even_j))
            def _():
                compute(j - 1, j, sr1, vr1, ob1, hr1, a1r, hs1)

            @pl.when(j + 1 < NCH)
            def _():
                even_n = lax.rem(j + 1, 2) == 0

                @pl.when(even_n)
                def _():
                    fire_idx(j + 1, hr0, sr0, a0r)

                @pl.when(jnp.logical_not(even_n))
                def _():
                    fire_idx(j + 1, hr1, sr1, a1r)

            return 0

        lax.fori_loop(0, NCH + 1, loop, 0)
        pltpu.make_async_copy(ob0, agg.at[hs0], sem_s).wait()
        pltpu.make_async_copy(ob1, agg.at[hs1], sem_s).wait()

    zero_agg()
    plsc.subcore_barrier()
    run_phase(er_hbm, head_hbm, kt_hbm, attn_hbm, ar0, ar1, False)
    plsc.subcore_barrier()
    pltpu.sync_copy(agg.at[pl.ds(off, 640), :],
                    pent_hbm.at[cid, pl.ds(off, 640), :])
    plsc.subcore_barrier()

    zero_agg()
    plsc.subcore_barrier()
    run_phase(ent_hbm, u_hbm, i_hbm, w_hbm, wr0, wr1, True)
    plsc.subcore_barrier()
    pltpu.sync_copy(agg.at[pl.ds(off, 640), :],
                    pusr_hbm.at[cid, pl.ds(off, 640), :])


def kernel(user_emb, entity_emb, edge_index, edge_type, inter_edge, inter_edge_w,
           relation_emb, W_Q):
    head2 = edge_index[0].reshape(NW, NCH, B)
    ktidx = ((edge_type - 1) % NREL) * N + edge_index[1]
    kt2 = ktidx.reshape(NW, NCH, B)
    u2 = inter_edge[0].reshape(NW, NCH, B)
    i2 = inter_edge[1].reshape(NW, NCH, B)
    w2 = inter_edge_w.reshape(NW, NCH, B)

    ent = entity_emb
    ent_res = entity_emb
    usr_res = user_emb
    for _ in range(2):
        T, KR3, ER3 = _tc_prep(ent, W_Q, relation_emb)
        KR = KR3.reshape(NREL * N, CH)
        ER = ER3.reshape(NREL * N, CH)
        scores, maxtab = _sc_scores(T, KR, head2, kt2)
        gmax = _tc_gmax(maxtab)
        sumtab = _sc_ssum(scores, head2, gmax)
        lden = _tc_lden(gmax, sumtab)
        attn = _sc_attn(scores, head2, lden)
        pent, pusr = _sc_agg(ER, kt2, head2, attn, ent, i2, u2, w2)
        ent, ent_res = _tc_merge(pent, ent_res)
        _, usr_res = _tc_merge(pusr, usr_res)
    return (ent_res, usr_res)
